# Initial kernel scaffold; baseline (speedup 1.0000x reference)
import jax
import jax.numpy as jnp
from jax.experimental import pallas as pl


def kernel(x, W1, b1, W2, b2, W3, b3, W4, b4, W5, b5, W6, b6):
    raise NotImplementedError("write your pallas kernel here")


# speedup vs baseline: 9.5599x; 9.5599x over previous
import functools

import jax
import jax.numpy as jnp
from jax.experimental import pallas as pl
from jax.experimental.pallas import tpu as pltpu

K = 10
N = 2048
ROW_TILE = 512


def _pointnet_kernel(x_nc_ref, x_cn_ref, w1_ref, b1_ref, w2_ref, b2_ref,
                     w3_ref, b3_ref, w4a_ref, w4b_ref, b4_ref, w5_ref,
                     b5_ref, w6_ref, b6_ref, out_ref, feat_ref):
    n = x_nc_ref.shape[1]
    for r0 in range(0, n, ROW_TILE):
        d = jnp.zeros((ROW_TILE, n), dtype=jnp.float32)
        for c in range(3):
            xr = x_nc_ref[0, pl.ds(r0, ROW_TILE), c:c + 1]
            xc = x_cn_ref[0, c:c + 1, :]
            diff = xr - xc
            d = d + diff * diff
        col_ids = jax.lax.broadcasted_iota(jnp.int32, (ROW_TILE, n), 1)
        ks = []
        for _ in range(K):
            m = jnp.min(d, axis=1, keepdims=True)
            am = jnp.argmin(d, axis=1).reshape(ROW_TILE, 1)
            ks.append(m)
            d = jnp.where(col_ids == am, jnp.inf, d)
        xr3 = x_nc_ref[0, pl.ds(r0, ROW_TILE), :]
        feat_ref[pl.ds(r0, ROW_TILE), :] = jnp.concatenate([xr3] + ks, axis=1)

    f = feat_ref[:, :]
    h = jnp.maximum(jnp.dot(f, w1_ref[:, :],
                            preferred_element_type=jnp.float32) + b1_ref[:, :], 0.0)
    x1 = jnp.maximum(jnp.dot(h, w2_ref[:, :],
                             preferred_element_type=jnp.float32) + b2_ref[:, :], 0.0)
    x2 = jnp.maximum(jnp.dot(x1, w3_ref[:, :],
                             preferred_element_type=jnp.float32) + b3_ref[:, :], 0.0)
    pool = jnp.mean(x2, axis=0, keepdims=True)
    o = jnp.dot(x1, w4a_ref[:, :], preferred_element_type=jnp.float32)
    o = o + jnp.dot(pool, w4b_ref[:, :], preferred_element_type=jnp.float32)
    o = jnp.maximum(o + b4_ref[:, :], 0.0)
    o = jnp.maximum(jnp.dot(o, w5_ref[:, :],
                            preferred_element_type=jnp.float32) + b5_ref[:, :], 0.0)
    logits = jnp.dot(o, w6_ref[:, :],
                     preferred_element_type=jnp.float32) + b6_ref[:, :]
    mx = jnp.max(logits, axis=1, keepdims=True)
    lse = mx + jnp.log(jnp.sum(jnp.exp(logits - mx), axis=1, keepdims=True))
    out_ref[0, :, :] = logits - lse


@jax.jit
def kernel(x, W1, b1, W2, b2, W3, b3, W4, b4, W5, b5, W6, b6):
    B, n, C = x.shape
    GF = W2.shape[0]
    x_cn = jnp.transpose(x, (0, 2, 1))
    w1t = jnp.transpose(W1)
    w2t = jnp.transpose(W2)
    w3t = jnp.transpose(W3)
    w4at = jnp.transpose(W4[:, :GF])
    w4bt = jnp.transpose(W4[:, GF:])
    w5t = jnp.transpose(W5)
    w6t = jnp.transpose(W6)
    biases = [b.reshape(1, -1) for b in (b1, b2, b3, b4, b5, b6)]

    full = lambda a: pl.BlockSpec(a.shape, lambda b: (0,) * a.ndim)
    in_specs = [
            pl.BlockSpec((1, n, C), lambda b: (b, 0, 0)),
            pl.BlockSpec((1, C, n), lambda b: (b, 0, 0)),
            full(w1t), full(biases[0]), full(w2t), full(biases[1]),
            full(w3t), full(biases[2]), full(w4at), full(w4bt),
            full(biases[3]), full(w5t), full(biases[4]), full(w6t),
            full(biases[5]),
    ]
    out = pl.pallas_call(
        _pointnet_kernel,
        grid=(B,),
        in_specs=in_specs,
        out_specs=pl.BlockSpec((1, n, 2), lambda b: (b, 0, 0)),
        out_shape=jax.ShapeDtypeStruct((B, n, 2), jnp.float32),
        scratch_shapes=[pltpu.VMEM((n, 3 + K), jnp.float32)],
        compiler_params=pltpu.CompilerParams(
            dimension_semantics=("arbitrary",),
        ),
    )(x, x_cn, w1t, biases[0], w2t, biases[1], w3t, biases[2],
      w4at, w4bt, biases[3], w5t, biases[4], w6t, biases[5])
    return out


# speedup vs baseline: 28.0391x; 2.9330x over previous
import functools

import jax
import jax.numpy as jnp
from jax.experimental import pallas as pl
from jax.experimental.pallas import tpu as pltpu

K = 10
N = 2048
ROW_TILE = 512


def _pointnet_kernel(x_nc_ref, x_cn_ref, w1_ref, b1_ref, w2_ref, b2_ref,
                     w3_ref, b3_ref, w4a_ref, w4b_ref, b4_ref, w5_ref,
                     b5_ref, w6_ref, b6_ref, out_ref, feat_ref):
    n = x_nc_ref.shape[1]
    for r0 in range(0, n, ROW_TILE):
        d = jnp.zeros((ROW_TILE, n), dtype=jnp.float32)
        for c in range(3):
            xr = x_nc_ref[0, pl.ds(r0, ROW_TILE), c:c + 1]
            xc = x_cn_ref[0, c:c + 1, :]
            diff = xr - xc
            d = d + diff * diff
        ks = []
        for _ in range(K):
            m = jnp.min(d, axis=1, keepdims=True)
            ks.append(m)
            d = jnp.where(d == m, jnp.inf, d)
        xr3 = x_nc_ref[0, pl.ds(r0, ROW_TILE), :]
        feat_ref[pl.ds(r0, ROW_TILE), :] = jnp.concatenate([xr3] + ks, axis=1)

    f = feat_ref[:, :]
    h = jnp.maximum(jnp.dot(f, w1_ref[:, :],
                            preferred_element_type=jnp.float32) + b1_ref[:, :], 0.0)
    x1 = jnp.maximum(jnp.dot(h, w2_ref[:, :],
                             preferred_element_type=jnp.float32) + b2_ref[:, :], 0.0)
    x2 = jnp.maximum(jnp.dot(x1, w3_ref[:, :],
                             preferred_element_type=jnp.float32) + b3_ref[:, :], 0.0)
    pool = jnp.mean(x2, axis=0, keepdims=True)
    o = jnp.dot(x1, w4a_ref[:, :], preferred_element_type=jnp.float32)
    o = o + jnp.dot(pool, w4b_ref[:, :], preferred_element_type=jnp.float32)
    o = jnp.maximum(o + b4_ref[:, :], 0.0)
    o = jnp.maximum(jnp.dot(o, w5_ref[:, :],
                            preferred_element_type=jnp.float32) + b5_ref[:, :], 0.0)
    logits = jnp.dot(o, w6_ref[:, :],
                     preferred_element_type=jnp.float32) + b6_ref[:, :]
    mx = jnp.max(logits, axis=1, keepdims=True)
    lse = mx + jnp.log(jnp.sum(jnp.exp(logits - mx), axis=1, keepdims=True))
    out_ref[0, :, :] = logits - lse


@jax.jit
def kernel(x, W1, b1, W2, b2, W3, b3, W4, b4, W5, b5, W6, b6):
    B, n, C = x.shape
    GF = W2.shape[0]
    x_cn = jnp.transpose(x, (0, 2, 1))
    w1t = jnp.transpose(W1)
    w2t = jnp.transpose(W2)
    w3t = jnp.transpose(W3)
    w4at = jnp.transpose(W4[:, :GF])
    w4bt = jnp.transpose(W4[:, GF:])
    w5t = jnp.transpose(W5)
    w6t = jnp.transpose(W6)
    biases = [b.reshape(1, -1) for b in (b1, b2, b3, b4, b5, b6)]

    full = lambda a: pl.BlockSpec(a.shape, lambda b: (0,) * a.ndim)
    in_specs = [
            pl.BlockSpec((1, n, C), lambda b: (b, 0, 0)),
            pl.BlockSpec((1, C, n), lambda b: (b, 0, 0)),
            full(w1t), full(biases[0]), full(w2t), full(biases[1]),
            full(w3t), full(biases[2]), full(w4at), full(w4bt),
            full(biases[3]), full(w5t), full(biases[4]), full(w6t),
            full(biases[5]),
    ]
    out = pl.pallas_call(
        _pointnet_kernel,
        grid=(B,),
        in_specs=in_specs,
        out_specs=pl.BlockSpec((1, n, 2), lambda b: (b, 0, 0)),
        out_shape=jax.ShapeDtypeStruct((B, n, 2), jnp.float32),
        scratch_shapes=[pltpu.VMEM((n, 3 + K), jnp.float32)],
        compiler_params=pltpu.CompilerParams(
            dimension_semantics=("parallel",),
        ),
    )(x, x_cn, w1t, biases[0], w2t, biases[1], w3t, biases[2],
      w4at, w4bt, biases[3], w5t, biases[4], w6t, biases[5])
    return out


# speedup vs baseline: 33.0463x; 1.1786x over previous
import functools

import jax
import jax.numpy as jnp
from jax.experimental import pallas as pl
from jax.experimental.pallas import tpu as pltpu

K = 10
N = 2048
ROW_TILE = 512


def _pointnet_kernel(x_nc_ref, x_cn_ref, w1_ref, b1_ref, w2_ref, b2_ref,
                     w3_ref, b3_ref, w4a_ref, w4b_ref, b4_ref, w5_ref,
                     b5_ref, w6_ref, b6_ref, out_ref, feat_ref):
    n = x_nc_ref.shape[1]
    xcn = x_cn_ref[0, :, :]
    ncol = jnp.sum(xcn * xcn, axis=0, keepdims=True)
    for r0 in range(0, n, ROW_TILE):
        xr3 = x_nc_ref[0, pl.ds(r0, ROW_TILE), :]
        nrow = jnp.sum(xr3 * xr3, axis=1, keepdims=True)
        g = jnp.dot(xr3, xcn, preferred_element_type=jnp.float32)
        d = nrow + ncol - 2.0 * g
        ks = []
        for t in range(K):
            m = jnp.min(d, axis=1, keepdims=True)
            ks.append(m)
            if t < K - 1:
                d = jnp.where(d == m, jnp.inf, d)
        xr3 = x_nc_ref[0, pl.ds(r0, ROW_TILE), :]
        feat_ref[pl.ds(r0, ROW_TILE), :] = jnp.concatenate([xr3] + ks, axis=1)

    f = feat_ref[:, :]
    h = jnp.maximum(jnp.dot(f, w1_ref[:, :],
                            preferred_element_type=jnp.float32) + b1_ref[:, :], 0.0)
    x1 = jnp.maximum(jnp.dot(h, w2_ref[:, :],
                             preferred_element_type=jnp.float32) + b2_ref[:, :], 0.0)
    x2 = jnp.maximum(jnp.dot(x1, w3_ref[:, :],
                             preferred_element_type=jnp.float32) + b3_ref[:, :], 0.0)
    pool = jnp.mean(x2, axis=0, keepdims=True)
    o = jnp.dot(x1, w4a_ref[:, :], preferred_element_type=jnp.float32)
    o = o + jnp.dot(pool, w4b_ref[:, :], preferred_element_type=jnp.float32)
    o = jnp.maximum(o + b4_ref[:, :], 0.0)
    o = jnp.maximum(jnp.dot(o, w5_ref[:, :],
                            preferred_element_type=jnp.float32) + b5_ref[:, :], 0.0)
    logits = jnp.dot(o, w6_ref[:, :],
                     preferred_element_type=jnp.float32) + b6_ref[:, :]
    mx = jnp.max(logits, axis=1, keepdims=True)
    lse = mx + jnp.log(jnp.sum(jnp.exp(logits - mx), axis=1, keepdims=True))
    out_ref[0, :, :] = logits - lse


@jax.jit
def kernel(x, W1, b1, W2, b2, W3, b3, W4, b4, W5, b5, W6, b6):
    B, n, C = x.shape
    GF = W2.shape[0]
    x_cn = jnp.transpose(x, (0, 2, 1))
    w1t = jnp.transpose(W1)
    w2t = jnp.transpose(W2)
    w3t = jnp.transpose(W3)
    w4at = jnp.transpose(W4[:, :GF])
    w4bt = jnp.transpose(W4[:, GF:])
    w5t = jnp.transpose(W5)
    w6t = jnp.transpose(W6)
    biases = [b.reshape(1, -1) for b in (b1, b2, b3, b4, b5, b6)]

    full = lambda a: pl.BlockSpec(a.shape, lambda b: (0,) * a.ndim)
    in_specs = [
            pl.BlockSpec((1, n, C), lambda b: (b, 0, 0)),
            pl.BlockSpec((1, C, n), lambda b: (b, 0, 0)),
            full(w1t), full(biases[0]), full(w2t), full(biases[1]),
            full(w3t), full(biases[2]), full(w4at), full(w4bt),
            full(biases[3]), full(w5t), full(biases[4]), full(w6t),
            full(biases[5]),
    ]
    out = pl.pallas_call(
        _pointnet_kernel,
        grid=(B,),
        in_specs=in_specs,
        out_specs=pl.BlockSpec((1, n, 2), lambda b: (b, 0, 0)),
        out_shape=jax.ShapeDtypeStruct((B, n, 2), jnp.float32),
        scratch_shapes=[pltpu.VMEM((n, 3 + K), jnp.float32)],
        compiler_params=pltpu.CompilerParams(
            dimension_semantics=("parallel",),
        ),
    )(x, x_cn, w1t, biases[0], w2t, biases[1], w3t, biases[2],
      w4at, w4bt, biases[3], w5t, biases[4], w6t, biases[5])
    return out


# speedup vs baseline: 33.3224x; 1.0084x over previous
import functools

import jax
import jax.numpy as jnp
from jax.experimental import pallas as pl
from jax.experimental.pallas import tpu as pltpu

K = 10
N = 2048
ROW_TILE = 512


def _pointnet_kernel(x_nc_ref, x_cn_ref, w1_ref, b1_ref, w2_ref, b2_ref,
                     w3_ref, b3_ref, w4a_ref, w4b_ref, b4_ref, w5_ref,
                     b5_ref, w6_ref, b6_ref, out_ref, feat_ref):
    n = x_nc_ref.shape[1]
    xcn = x_cn_ref[0, :, :]
    ncol = jnp.sum(xcn * xcn, axis=0, keepdims=True)
    for r0 in range(0, n, ROW_TILE):
        xr3 = x_nc_ref[0, pl.ds(r0, ROW_TILE), :]
        nrow = jnp.sum(xr3 * xr3, axis=1, keepdims=True)
        g = jnp.dot(xr3, xcn, preferred_element_type=jnp.float32)
        d = nrow + ncol - 2.0 * g
        m = jnp.min(d, axis=1, keepdims=True)
        ks = [m]
        for _ in range(K - 1):
            m = jnp.min(jnp.where(d > m, d, jnp.inf), axis=1, keepdims=True)
            ks.append(m)
        xr3 = x_nc_ref[0, pl.ds(r0, ROW_TILE), :]
        feat_ref[pl.ds(r0, ROW_TILE), :] = jnp.concatenate([xr3] + ks, axis=1)

    f = feat_ref[:, :]
    h = jnp.maximum(jnp.dot(f, w1_ref[:, :],
                            preferred_element_type=jnp.float32) + b1_ref[:, :], 0.0)
    x1 = jnp.maximum(jnp.dot(h, w2_ref[:, :],
                             preferred_element_type=jnp.float32) + b2_ref[:, :], 0.0)
    x2 = jnp.maximum(jnp.dot(x1, w3_ref[:, :],
                             preferred_element_type=jnp.float32) + b3_ref[:, :], 0.0)
    pool = jnp.mean(x2, axis=0, keepdims=True)
    o = jnp.dot(x1, w4a_ref[:, :], preferred_element_type=jnp.float32)
    o = o + jnp.dot(pool, w4b_ref[:, :], preferred_element_type=jnp.float32)
    o = jnp.maximum(o + b4_ref[:, :], 0.0)
    o = jnp.maximum(jnp.dot(o, w5_ref[:, :],
                            preferred_element_type=jnp.float32) + b5_ref[:, :], 0.0)
    logits = jnp.dot(o, w6_ref[:, :],
                     preferred_element_type=jnp.float32) + b6_ref[:, :]
    mx = jnp.max(logits, axis=1, keepdims=True)
    lse = mx + jnp.log(jnp.sum(jnp.exp(logits - mx), axis=1, keepdims=True))
    out_ref[0, :, :] = logits - lse


@jax.jit
def kernel(x, W1, b1, W2, b2, W3, b3, W4, b4, W5, b5, W6, b6):
    B, n, C = x.shape
    GF = W2.shape[0]
    x_cn = jnp.transpose(x, (0, 2, 1))
    w1t = jnp.transpose(W1)
    w2t = jnp.transpose(W2)
    w3t = jnp.transpose(W3)
    w4at = jnp.transpose(W4[:, :GF])
    w4bt = jnp.transpose(W4[:, GF:])
    w5t = jnp.transpose(W5)
    w6t = jnp.transpose(W6)
    biases = [b.reshape(1, -1) for b in (b1, b2, b3, b4, b5, b6)]

    full = lambda a: pl.BlockSpec(a.shape, lambda b: (0,) * a.ndim)
    in_specs = [
            pl.BlockSpec((1, n, C), lambda b: (b, 0, 0)),
            pl.BlockSpec((1, C, n), lambda b: (b, 0, 0)),
            full(w1t), full(biases[0]), full(w2t), full(biases[1]),
            full(w3t), full(biases[2]), full(w4at), full(w4bt),
            full(biases[3]), full(w5t), full(biases[4]), full(w6t),
            full(biases[5]),
    ]
    out = pl.pallas_call(
        _pointnet_kernel,
        grid=(B,),
        in_specs=in_specs,
        out_specs=pl.BlockSpec((1, n, 2), lambda b: (b, 0, 0)),
        out_shape=jax.ShapeDtypeStruct((B, n, 2), jnp.float32),
        scratch_shapes=[pltpu.VMEM((n, 3 + K), jnp.float32)],
        compiler_params=pltpu.CompilerParams(
            dimension_semantics=("parallel",),
        ),
    )(x, x_cn, w1t, biases[0], w2t, biases[1], w3t, biases[2],
      w4at, w4bt, biases[3], w5t, biases[4], w6t, biases[5])
    return out
